# baseline (speedup 1.0000x reference)
import functools

import jax
import jax.numpy as jnp
from jax import lax
from jax.experimental import pallas as pl
from jax.experimental.pallas import tpu as pltpu
from jax.experimental.pallas import tpu_sc as plsc

VOCAB = 1000000
EMBED_DIM = 64

NUM_CORES = 2
NUM_SUBCORES = 16
NUM_WORKERS = NUM_CORES * NUM_SUBCORES

FAT_ROWS = 500032
N_BLOCKS = 7812
TAIL_V0 = 999936
TAIL_FAT0 = TAIL_V0 // 2

_IOTA16 = None


def _iota16():
    return lax.iota(jnp.int32, 16)


def _splat(val):
    return jnp.full((16,), val, jnp.int32)


def _make_pack():
    mesh = plsc.VectorSubcoreMesh(
        core_axis_name="c", subcore_axis_name="s",
        num_cores=NUM_CORES, num_subcores=NUM_SUBCORES)

    @functools.partial(
        pl.kernel,
        out_type=jax.ShapeDtypeStruct((FAT_ROWS, 128), jnp.float32),
        mesh=mesh,
        scratch_types=[
            pltpu.VMEM((2, 64, 128), jnp.float32),
            pltpu.VMEM((2, 64, 128), jnp.float32),
            pltpu.SemaphoreType.DMA,
            pltpu.SemaphoreType.DMA,
            pltpu.SemaphoreType.DMA,
            pltpu.SemaphoreType.DMA,
        ],
        compiler_params=pltpu.CompilerParams(use_tc_tiling_on_sc=True, needs_layout_passes=False),
    )
    def pack(tt_hbm, tail_hbm, fat_hbm, a3, b3, lsem0, lsem1, ssem0, ssem1):
        w = lax.axis_index("s") * NUM_CORES + lax.axis_index("c")
        n = (N_BLOCKS - w + NUM_WORKERS - 1) // NUM_WORKERS
        iota = _iota16()

        @pl.when(w == 5)
        def _():
            pltpu.sync_copy(tail_hbm, b3.at[1, pl.ds(0, 32)])
            pltpu.sync_copy(b3.at[1, pl.ds(0, 32)],
                            fat_hbm.at[pl.ds(TAIL_FAT0, 32)])

        def fire_load(i, buf_is_0):
            b = w + NUM_WORKERS * i
            v0 = b * 128

            @pl.when(buf_is_0)
            def _():
                pltpu.async_copy(tt_hbm.at[:, pl.ds(v0, 128)], a3.at[0], lsem0)

            @pl.when(jnp.logical_not(buf_is_0))
            def _():
                pltpu.async_copy(tt_hbm.at[:, pl.ds(v0, 128)], a3.at[1], lsem1)

        pltpu.async_copy(tt_hbm.at[:, pl.ds(w * 128, 128)], a3.at[0], lsem0)

        def step(i, carry):
            buf = i % 2
            buf_is_0 = buf == 0
            b = w + NUM_WORKERS * i
            fat0 = b * 64

            @pl.when(i + 1 < n)
            def _():
                fire_load(i + 1, jnp.logical_not(buf_is_0))

            @pl.when(buf_is_0)
            def _():
                pltpu.make_async_copy(
                    tt_hbm.at[:, pl.ds(0, 128)], a3.at[0], lsem0).wait()

            @pl.when(jnp.logical_not(buf_is_0))
            def _():
                pltpu.make_async_copy(
                    tt_hbm.at[:, pl.ds(0, 128)], a3.at[1], lsem1).wait()

            @pl.when(jnp.logical_and(i >= 2, buf_is_0))
            def _():
                pltpu.make_async_copy(
                    b3.at[0], fat_hbm.at[pl.ds(0, 64)], ssem0).wait()

            @pl.when(jnp.logical_and(i >= 2, jnp.logical_not(buf_is_0)))
            def _():
                pltpu.make_async_copy(
                    b3.at[1], fat_hbm.at[pl.ds(0, 64)], ssem1).wait()

            bufv = _splat(buf)

            def frow(f, c2):
                for q in range(8):
                    h = q // 4
                    row_idx = iota + ((q % 4) * 16)
                    col_idx = _splat(2 * f + h)
                    g = plsc.load_gather(a3, [bufv, row_idx, col_idx])
                    b3[buf, f, pl.ds(q * 16, 16)] = g
                return c2

            lax.fori_loop(0, 64, frow, 0, unroll=2)

            @pl.when(buf_is_0)
            def _():
                pltpu.async_copy(b3.at[0], fat_hbm.at[pl.ds(fat0, 64)], ssem0)

            @pl.when(jnp.logical_not(buf_is_0))
            def _():
                pltpu.async_copy(b3.at[1], fat_hbm.at[pl.ds(fat0, 64)], ssem1)

            return carry

        lax.fori_loop(0, n, step, 0)

        @pl.when(n >= 2)
        def _():
            pltpu.make_async_copy(
                b3.at[0], fat_hbm.at[pl.ds(0, 64)], ssem0).wait()

        @pl.when(n >= 2)
        def _():
            pltpu.make_async_copy(
                b3.at[1], fat_hbm.at[pl.ds(0, 64)], ssem1).wait()

    return pack


def _make_gather():
    mesh = plsc.VectorSubcoreMesh(
        core_axis_name="c", subcore_axis_name="s",
        num_cores=NUM_CORES, num_subcores=NUM_SUBCORES)

    @functools.partial(
        pl.kernel,
        out_type=jax.ShapeDtypeStruct((200, EMBED_DIM, 4096), jnp.float32),
        mesh=mesh,
        scratch_types=[
            pltpu.VMEM((2, 8, 128), jnp.int32),
            pltpu.VMEM((2, 8, 128), jnp.int32),
            pltpu.VMEM((2, 8, 128), jnp.int32),
            pltpu.VMEM((256, 128), jnp.float32),
            pltpu.VMEM((256, 128), jnp.float32),
            pltpu.VMEM((2, 64, 128), jnp.float32),
            pltpu.SemaphoreType.DMA,
            pltpu.SemaphoreType.DMA,
            pltpu.SemaphoreType.DMA,
            pltpu.SemaphoreType.DMA,
        ],
        compiler_params=pltpu.CompilerParams(use_tc_tiling_on_sc=True, needs_layout_passes=False),
    )
    def gather(xt_hbm, fat_hbm, out_hbm, idx3, fid3, par3,
               fb0, fb1, stg, gsem0, gsem1, osem0, osem1):
        w = lax.axis_index("s") * NUM_CORES + lax.axis_index("c")
        b0 = w * 128
        iota = _iota16()

        def prep(o):
            p = o % 2
            pltpu.sync_copy(xt_hbm.at[pl.ds(o * 8, 8), pl.ds(b0, 128)],
                            idx3.at[p])

            def chunk(c, carry):
                r = c // 8
                k = c % 8
                v = idx3[p, r, pl.ds(k * 16, 16)]
                fid3[p, r, pl.ds(k * 16, 16)] = lax.shift_right_logical(v, 1)
                par3[p, r, pl.ds(k * 16, 16)] = lax.shift_left(
                    lax.bitwise_and(v, 1), 6)
                return carry

            lax.fori_loop(0, 64, chunk, 0, unroll=4)

        def fire(q, buf0):
            o = q // 4
            p = o % 2
            r0 = 2 * (q % 4)
            for j in range(2):
                @pl.when(buf0)
                def _():
                    pltpu.async_copy(fat_hbm.at[fid3.at[p, r0 + j]],
                                     fb0.at[pl.ds(j * 128, 128)], gsem0)

                @pl.when(jnp.logical_not(buf0))
                def _():
                    pltpu.async_copy(fat_hbm.at[fid3.at[p, r0 + j]],
                                     fb1.at[pl.ds(j * 128, 128)], gsem1)

        def transpose_pair(q, fbuf, fb):
            o = q // 4
            p = o % 2
            for sloc in range(2):
                s = 2 * q + sloc
                rloc = 2 * (q % 4) + sloc

                @pl.when(q >= 1)
                def _():
                    sem = osem0 if sloc == 0 else osem1
                    pltpu.make_async_copy(
                        stg.at[sloc],
                        out_hbm.at[0, :, pl.ds(b0, 128)], sem).wait()

                par_k = [par3[p, rloc, pl.ds(k * 16, 16)] for k in range(8)]
                rows_k = [iota + (sloc * 128 + k * 16) for k in range(8)]

                def drow(d, carry):
                    for k in range(8):
                        col = par_k[k] + d
                        g = plsc.load_gather(fb, [rows_k[k], col])
                        stg[sloc, d, pl.ds(k * 16, 16)] = g
                    return carry

                lax.fori_loop(0, 64, drow, 0, unroll=2)

                sem = osem0 if sloc == 0 else osem1
                pltpu.async_copy(stg.at[sloc],
                                 out_hbm.at[s, :, pl.ds(b0, 128)], sem)

        prep(0)
        fire(0, jnp.bool_(True))

        def step(t, carry):
            for half in range(2):
                q = 2 * t + half
                buf0 = half == 0
                nxt = q + 1

                @pl.when(nxt < 100)
                def _():
                    @pl.when(nxt % 4 == 0)
                    def _():
                        prep(nxt // 4)
                    fire(nxt, jnp.bool_(not buf0))

                fb = fb0 if buf0 else fb1
                gsem = gsem0 if buf0 else gsem1
                for j in range(2):
                    pltpu.make_async_copy(
                        fat_hbm.at[fid3.at[0, 0]],
                        fb.at[pl.ds(j * 128, 128)], gsem).wait()

                transpose_pair(q, 0, fb)
            return carry

        lax.fori_loop(0, 50, step, 0)

        pltpu.make_async_copy(
            stg.at[0], out_hbm.at[0, :, pl.ds(b0, 128)], osem0).wait()
        pltpu.make_async_copy(
            stg.at[1], out_hbm.at[0, :, pl.ds(b0, 128)], osem1).wait()

    return gather


def kernel(x, table):
    tail_fat = jnp.concatenate(
        [table[TAIL_V0::2], table[TAIL_V0 + 1::2]], axis=1)
    fat = _make_pack()(table.T, tail_fat)
    out3 = _make_gather()(x.T, fat)
    return out3.transpose(2, 0, 1)


# speedup vs baseline: 1.2978x; 1.2978x over previous
import functools

import jax
import jax.numpy as jnp
from jax import lax
from jax.experimental import pallas as pl
from jax.experimental.pallas import tpu as pltpu
from jax.experimental.pallas import tpu_sc as plsc

VOCAB = 1000000
EMBED_DIM = 64

NUM_CORES = 2
NUM_SUBCORES = 16
NUM_WORKERS = NUM_CORES * NUM_SUBCORES

FAT_ROWS = 500032
N_BLOCKS = 7812
TAIL_V0 = 999936
TAIL_FAT0 = TAIL_V0 // 2

_IOTA16 = None


def _iota16():
    return lax.iota(jnp.int32, 16)


def _splat(val):
    return jnp.full((16,), val, jnp.int32)


def _make_pack():
    mesh = plsc.VectorSubcoreMesh(
        core_axis_name="c", subcore_axis_name="s",
        num_cores=NUM_CORES, num_subcores=NUM_SUBCORES)

    @functools.partial(
        pl.kernel,
        out_type=jax.ShapeDtypeStruct((FAT_ROWS, 128), jnp.float32),
        mesh=mesh,
        scratch_types=[
            pltpu.VMEM((2, 64, 128), jnp.float32),
            pltpu.VMEM((2, 64, 128), jnp.float32),
            pltpu.SemaphoreType.DMA,
            pltpu.SemaphoreType.DMA,
            pltpu.SemaphoreType.DMA,
            pltpu.SemaphoreType.DMA,
        ],
        compiler_params=pltpu.CompilerParams(use_tc_tiling_on_sc=True, needs_layout_passes=False),
    )
    def pack(tt_hbm, tail_hbm, fat_hbm, a3, b3, lsem0, lsem1, ssem0, ssem1):
        w = lax.axis_index("s") * NUM_CORES + lax.axis_index("c")
        n = (N_BLOCKS - w + NUM_WORKERS - 1) // NUM_WORKERS
        iota = _iota16()

        @pl.when(w == 5)
        def _():
            pltpu.sync_copy(tail_hbm, b3.at[1, pl.ds(0, 32)])
            pltpu.sync_copy(b3.at[1, pl.ds(0, 32)],
                            fat_hbm.at[pl.ds(TAIL_FAT0, 32)])

        def fire_load(i, buf_is_0):
            b = w + NUM_WORKERS * i
            v0 = b * 128

            @pl.when(buf_is_0)
            def _():
                pltpu.async_copy(tt_hbm.at[:, pl.ds(v0, 128)], a3.at[0], lsem0)

            @pl.when(jnp.logical_not(buf_is_0))
            def _():
                pltpu.async_copy(tt_hbm.at[:, pl.ds(v0, 128)], a3.at[1], lsem1)

        pltpu.async_copy(tt_hbm.at[:, pl.ds(w * 128, 128)], a3.at[0], lsem0)

        def step(i, carry):
            buf = i % 2
            buf_is_0 = buf == 0
            b = w + NUM_WORKERS * i
            fat0 = b * 64

            @pl.when(i + 1 < n)
            def _():
                fire_load(i + 1, jnp.logical_not(buf_is_0))

            @pl.when(buf_is_0)
            def _():
                pltpu.make_async_copy(
                    tt_hbm.at[:, pl.ds(0, 128)], a3.at[0], lsem0).wait()

            @pl.when(jnp.logical_not(buf_is_0))
            def _():
                pltpu.make_async_copy(
                    tt_hbm.at[:, pl.ds(0, 128)], a3.at[1], lsem1).wait()

            @pl.when(jnp.logical_and(i >= 2, buf_is_0))
            def _():
                pltpu.make_async_copy(
                    b3.at[0], fat_hbm.at[pl.ds(0, 64)], ssem0).wait()

            @pl.when(jnp.logical_and(i >= 2, jnp.logical_not(buf_is_0)))
            def _():
                pltpu.make_async_copy(
                    b3.at[1], fat_hbm.at[pl.ds(0, 64)], ssem1).wait()

            bufv = _splat(buf)

            def frow(f, c2):
                gs = []
                for q in range(8):
                    h = q // 4
                    row_idx = iota + ((q % 4) * 16)
                    col_idx = _splat(2 * f + h)
                    gs.append(plsc.load_gather(a3, [bufv, row_idx, col_idx]))
                for q in range(8):
                    b3[buf, f, pl.ds(q * 16, 16)] = gs[q]
                return c2

            lax.fori_loop(0, 64, frow, 0, unroll=4)

            @pl.when(buf_is_0)
            def _():
                pltpu.async_copy(b3.at[0], fat_hbm.at[pl.ds(fat0, 64)], ssem0)

            @pl.when(jnp.logical_not(buf_is_0))
            def _():
                pltpu.async_copy(b3.at[1], fat_hbm.at[pl.ds(fat0, 64)], ssem1)

            return carry

        lax.fori_loop(0, n, step, 0)

        @pl.when(n >= 2)
        def _():
            pltpu.make_async_copy(
                b3.at[0], fat_hbm.at[pl.ds(0, 64)], ssem0).wait()

        @pl.when(n >= 2)
        def _():
            pltpu.make_async_copy(
                b3.at[1], fat_hbm.at[pl.ds(0, 64)], ssem1).wait()

    return pack


def _make_gather():
    mesh = plsc.VectorSubcoreMesh(
        core_axis_name="c", subcore_axis_name="s",
        num_cores=NUM_CORES, num_subcores=NUM_SUBCORES)

    @functools.partial(
        pl.kernel,
        out_type=jax.ShapeDtypeStruct((200, EMBED_DIM, 4096), jnp.float32),
        mesh=mesh,
        scratch_types=[
            pltpu.VMEM((2, 8, 128), jnp.int32),
            pltpu.VMEM((2, 8, 128), jnp.int32),
            pltpu.VMEM((2, 8, 128), jnp.int32),
            pltpu.VMEM((256, 128), jnp.float32),
            pltpu.VMEM((256, 128), jnp.float32),
            pltpu.VMEM((2, 64, 128), jnp.float32),
            pltpu.SemaphoreType.DMA,
            pltpu.SemaphoreType.DMA,
            pltpu.SemaphoreType.DMA,
            pltpu.SemaphoreType.DMA,
        ],
        compiler_params=pltpu.CompilerParams(use_tc_tiling_on_sc=True, needs_layout_passes=False),
    )
    def gather(xt_hbm, fat_hbm, out_hbm, idx3, fid3, par3,
               fb0, fb1, stg, gsem0, gsem1, osem0, osem1):
        w = lax.axis_index("s") * NUM_CORES + lax.axis_index("c")
        b0 = w * 128
        iota = _iota16()

        def prep(o):
            p = o % 2
            pltpu.sync_copy(xt_hbm.at[pl.ds(o * 8, 8), pl.ds(b0, 128)],
                            idx3.at[p])

            def chunk(c, carry):
                r = c // 8
                k = c % 8
                v = idx3[p, r, pl.ds(k * 16, 16)]
                fid3[p, r, pl.ds(k * 16, 16)] = lax.shift_right_logical(v, 1)
                par3[p, r, pl.ds(k * 16, 16)] = lax.shift_left(
                    lax.bitwise_and(v, 1), 6)
                return carry

            lax.fori_loop(0, 64, chunk, 0, unroll=4)

        def fire(q, buf0):
            o = q // 4
            p = o % 2
            r0 = 2 * (q % 4)
            for j in range(2):
                @pl.when(buf0)
                def _():
                    pltpu.async_copy(fat_hbm.at[fid3.at[p, r0 + j]],
                                     fb0.at[pl.ds(j * 128, 128)], gsem0)

                @pl.when(jnp.logical_not(buf0))
                def _():
                    pltpu.async_copy(fat_hbm.at[fid3.at[p, r0 + j]],
                                     fb1.at[pl.ds(j * 128, 128)], gsem1)

        def transpose_pair(q, fbuf, fb):
            o = q // 4
            p = o % 2
            for sloc in range(2):
                s = 2 * q + sloc
                rloc = 2 * (q % 4) + sloc

                @pl.when(q >= 1)
                def _():
                    sem = osem0 if sloc == 0 else osem1
                    pltpu.make_async_copy(
                        stg.at[sloc],
                        out_hbm.at[0, :, pl.ds(b0, 128)], sem).wait()

                par_k = [par3[p, rloc, pl.ds(k * 16, 16)] for k in range(8)]
                rows_k = [iota + (sloc * 128 + k * 16) for k in range(8)]

                def drow(d, carry):
                    gs = []
                    for k in range(8):
                        col = par_k[k] + d
                        gs.append(plsc.load_gather(fb, [rows_k[k], col]))
                    for k in range(8):
                        stg[sloc, d, pl.ds(k * 16, 16)] = gs[k]
                    return carry

                lax.fori_loop(0, 64, drow, 0, unroll=4)

                sem = osem0 if sloc == 0 else osem1
                pltpu.async_copy(stg.at[sloc],
                                 out_hbm.at[s, :, pl.ds(b0, 128)], sem)

        prep(0)
        fire(0, jnp.bool_(True))

        def step(t, carry):
            for half in range(2):
                q = 2 * t + half
                buf0 = half == 0
                nxt = q + 1

                @pl.when(nxt < 100)
                def _():
                    @pl.when(nxt % 4 == 0)
                    def _():
                        prep(nxt // 4)
                    fire(nxt, jnp.bool_(not buf0))

                fb = fb0 if buf0 else fb1
                gsem = gsem0 if buf0 else gsem1
                for j in range(2):
                    pltpu.make_async_copy(
                        fat_hbm.at[fid3.at[0, 0]],
                        fb.at[pl.ds(j * 128, 128)], gsem).wait()

                transpose_pair(q, 0, fb)
            return carry

        lax.fori_loop(0, 50, step, 0)

        pltpu.make_async_copy(
            stg.at[0], out_hbm.at[0, :, pl.ds(b0, 128)], osem0).wait()
        pltpu.make_async_copy(
            stg.at[1], out_hbm.at[0, :, pl.ds(b0, 128)], osem1).wait()

    return gather


def kernel(x, table):
    tail_fat = jnp.concatenate(
        [table[TAIL_V0::2], table[TAIL_V0 + 1::2]], axis=1)
    fat = _make_pack()(table.T, tail_fat)
    out3 = _make_gather()(x.T, fat)
    return out3.transpose(2, 0, 1)


# speedup vs baseline: 2.1982x; 1.6938x over previous
import functools

import jax
import jax.numpy as jnp
from jax import lax
from jax.experimental import pallas as pl
from jax.experimental.pallas import tpu as pltpu
from jax.experimental.pallas import tpu_sc as plsc

VOCAB = 1000000
EMBED_DIM = 64

NUM_CORES = 2
NUM_SUBCORES = 16
NUM_WORKERS = NUM_CORES * NUM_SUBCORES

IDX_MINOR = 128
K = 8
GROUP = K * IDX_MINOR


def _make_kernel(n_lookups):
    assert n_lookups % (NUM_WORKERS * GROUP) == 0
    rows_per_worker = n_lookups // (NUM_WORKERS * IDX_MINOR)
    groups = rows_per_worker // K

    mesh = plsc.VectorSubcoreMesh(
        core_axis_name="c", subcore_axis_name="s",
        num_cores=NUM_CORES, num_subcores=NUM_SUBCORES)

    @functools.partial(
        pl.kernel,
        out_type=jax.ShapeDtypeStruct((n_lookups, EMBED_DIM), jnp.float32),
        mesh=mesh,
        scratch_types=[
            pltpu.VMEM((K, IDX_MINOR), jnp.int32),
            pltpu.VMEM((GROUP, EMBED_DIM), jnp.float32),
            pltpu.SemaphoreType.DMA,
        ],
        compiler_params=pltpu.CompilerParams(use_tc_tiling_on_sc=False),
    )
    def body(x_hbm, table_hbm, out_hbm, idx_v, rows_v, sem):
        wid = lax.axis_index("s") * NUM_CORES + lax.axis_index("c")
        row_base = wid * rows_per_worker

        def group(g, carry):
            row0 = row_base + g * K
            pltpu.sync_copy(x_hbm.at[pl.ds(row0, K)], idx_v)
            copies = []
            for j in range(K):
                copies.append(pltpu.async_copy(
                    table_hbm.at[idx_v.at[j]],
                    rows_v.at[pl.ds(j * IDX_MINOR, IDX_MINOR)],
                    sem))
            for c in copies:
                c.wait()
            pltpu.sync_copy(rows_v,
                            out_hbm.at[pl.ds(row0 * IDX_MINOR, GROUP)])
            return carry

        lax.fori_loop(0, groups, group, 0)

    return body


def kernel(x, table):
    b, s = x.shape
    n = b * s
    x_flat = x.reshape(n // IDX_MINOR, IDX_MINOR)
    out = _make_kernel(n)(x_flat, table)
    return out.reshape(b, s, EMBED_DIM)


# speedup vs baseline: 2.2175x; 1.0087x over previous
import functools

import jax
import jax.numpy as jnp
from jax import lax
from jax.experimental import pallas as pl
from jax.experimental.pallas import tpu as pltpu
from jax.experimental.pallas import tpu_sc as plsc

VOCAB = 1000000
EMBED_DIM = 64

NUM_CORES = 2
NUM_SUBCORES = 16
NUM_WORKERS = NUM_CORES * NUM_SUBCORES

IDX_MINOR = 128
K = 8
GROUP = K * IDX_MINOR


def _make_kernel(n_lookups):
    assert n_lookups % (NUM_WORKERS * GROUP) == 0
    rows_per_worker = n_lookups // (NUM_WORKERS * IDX_MINOR)
    groups = rows_per_worker // K

    mesh = plsc.VectorSubcoreMesh(
        core_axis_name="c", subcore_axis_name="s",
        num_cores=NUM_CORES, num_subcores=NUM_SUBCORES)

    halves = groups * 2
    HK = K // 2
    HALF = HK * IDX_MINOR

    @functools.partial(
        pl.kernel,
        out_type=jax.ShapeDtypeStruct((n_lookups, EMBED_DIM), jnp.float32),
        mesh=mesh,
        scratch_types=[
            pltpu.VMEM((2, K, IDX_MINOR), jnp.int32),
            pltpu.VMEM((3, HALF, EMBED_DIM), jnp.float32),
            pltpu.SemaphoreType.DMA((3,)),
            pltpu.SemaphoreType.DMA((3,)),
        ],
        compiler_params=pltpu.CompilerParams(use_tc_tiling_on_sc=False),
    )
    def body(x_hbm, table_hbm, out_hbm, idx3, rows3, gsem, wsem):
        wid = lax.axis_index("s") * NUM_CORES + lax.axis_index("c")
        row_base = wid * rows_per_worker

        def load_idx(g):
            pltpu.sync_copy(x_hbm.at[pl.ds(row_base + g * K, K)],
                            idx3.at[g % 2])

        def fire(h, slot):
            p = (h // 2) % 2
            r0 = (h % 2) * HK
            for j in range(HK):
                pltpu.async_copy(
                    table_hbm.at[idx3.at[p, r0 + j]],
                    rows3.at[slot, pl.ds(j * IDX_MINOR, IDX_MINOR)],
                    gsem.at[slot])

        load_idx(0)
        fire(0, 0)
        fire(1, 1)

        def step(h, carry):
            cur = h % 3
            nxt = (h + 2) % 3

            for j in range(HK):
                pltpu.make_async_copy(
                    table_hbm.at[idx3.at[0, 0]],
                    rows3.at[cur, pl.ds(j * IDX_MINOR, IDX_MINOR)],
                    gsem.at[cur]).wait()

            @pl.when(h + 2 < halves)
            def _():
                @pl.when((h + 2) % 2 == 0)
                def _():
                    load_idx((h + 2) // 2)

                @pl.when(h >= 1)
                def _():
                    pltpu.make_async_copy(
                        rows3.at[nxt], out_hbm.at[pl.ds(0, HALF)],
                        wsem.at[nxt]).wait()

                fire(h + 2, nxt)

            pltpu.async_copy(
                rows3.at[cur],
                out_hbm.at[pl.ds(row_base * IDX_MINOR + h * HALF, HALF)],
                wsem.at[cur])
            return carry

        lax.fori_loop(0, halves, step, 0)

        for slot in range(3):
            pltpu.make_async_copy(
                rows3.at[slot], out_hbm.at[pl.ds(0, HALF)],
                wsem.at[slot]).wait()

    return body


def kernel(x, table):
    b, s = x.shape
    n = b * s
    x_flat = x.reshape(n // IDX_MINOR, IDX_MINOR)
    out = _make_kernel(n)(x_flat, table)
    return out.reshape(b, s, EMBED_DIM)


# speedup vs baseline: 2.2311x; 1.0061x over previous
import functools

import jax
import jax.numpy as jnp
from jax import lax
from jax.experimental import pallas as pl
from jax.experimental.pallas import tpu as pltpu
from jax.experimental.pallas import tpu_sc as plsc

VOCAB = 1000000
EMBED_DIM = 64

NUM_CORES = 2
NUM_SUBCORES = 16
NUM_WORKERS = NUM_CORES * NUM_SUBCORES

IDX_MINOR = 128
K = 8
GROUP = K * IDX_MINOR


def _make_kernel(n_lookups):
    assert n_lookups % (NUM_WORKERS * GROUP) == 0
    rows_per_worker = n_lookups // (NUM_WORKERS * IDX_MINOR)
    groups = rows_per_worker // K

    mesh = plsc.VectorSubcoreMesh(
        core_axis_name="c", subcore_axis_name="s",
        num_cores=NUM_CORES, num_subcores=NUM_SUBCORES)

    halves = groups * 2
    HK = K // 2
    HALF = HK * IDX_MINOR

    @functools.partial(
        pl.kernel,
        out_type=jax.ShapeDtypeStruct((n_lookups, EMBED_DIM), jnp.float32),
        mesh=mesh,
        scratch_types=[
            pltpu.VMEM((2, K, IDX_MINOR), jnp.int32),
            pltpu.VMEM((3, HALF, EMBED_DIM), jnp.float32),
            pltpu.SemaphoreType.DMA((3,)),
            pltpu.SemaphoreType.DMA((3,)),
            pltpu.SemaphoreType.DMA((2,)),
        ],
        compiler_params=pltpu.CompilerParams(use_tc_tiling_on_sc=False),
    )
    def body(x_hbm, table_hbm, out_hbm, idx3, rows3, gsem, wsem, isem):
        wid = lax.axis_index("s") * NUM_CORES + lax.axis_index("c")
        row_base = wid * rows_per_worker

        def load_idx(g):
            pltpu.async_copy(x_hbm.at[pl.ds(row_base + g * K, K)],
                             idx3.at[g % 2], isem.at[g % 2])

        def wait_idx(g):
            pltpu.make_async_copy(x_hbm.at[pl.ds(0, K)], idx3.at[g % 2],
                                  isem.at[g % 2]).wait()

        def fire(h, slot):
            p = (h // 2) % 2
            r0 = (h % 2) * HK
            for j in range(HK):
                pltpu.async_copy(
                    table_hbm.at[idx3.at[p, r0 + j]],
                    rows3.at[slot, pl.ds(j * IDX_MINOR, IDX_MINOR)],
                    gsem.at[slot])

        load_idx(0)
        wait_idx(0)
        fire(0, 0)
        fire(1, 1)
        load_idx(1)

        def step(h, carry):
            cur = h % 3
            nxt = (h + 2) % 3

            for j in range(HK):
                pltpu.make_async_copy(
                    table_hbm.at[idx3.at[0, 0]],
                    rows3.at[cur, pl.ds(j * IDX_MINOR, IDX_MINOR)],
                    gsem.at[cur]).wait()

            @pl.when(h + 2 < halves)
            def _():
                @pl.when((h + 2) % 2 == 0)
                def _():
                    wait_idx((h + 2) // 2)

                @pl.when(h >= 1)
                def _():
                    pltpu.make_async_copy(
                        rows3.at[nxt], out_hbm.at[pl.ds(0, HALF)],
                        wsem.at[nxt]).wait()

                fire(h + 2, nxt)

                @pl.when(jnp.logical_and((h + 2) % 2 == 1,
                                         (h + 3) // 2 < groups))
                def _():
                    load_idx((h + 3) // 2)

            pltpu.async_copy(
                rows3.at[cur],
                out_hbm.at[pl.ds(row_base * IDX_MINOR + h * HALF, HALF)],
                wsem.at[cur])
            return carry

        lax.fori_loop(0, halves, step, 0)

        for slot in range(3):
            pltpu.make_async_copy(
                rows3.at[slot], out_hbm.at[pl.ds(0, HALF)],
                wsem.at[slot]).wait()

    return body


def kernel(x, table):
    b, s = x.shape
    n = b * s
    x_flat = x.reshape(n // IDX_MINOR, IDX_MINOR)
    out = _make_kernel(n)(x_flat, table)
    return out.reshape(b, s, EMBED_DIM)
